# baseline (speedup 1.0000x reference)
import functools

import jax
import jax.numpy as jnp
from jax import lax
from jax.experimental import pallas as pl
from jax.experimental.pallas import tpu as pltpu
from jax.experimental.pallas import tpu_sc as plsc

NU = 50000
NI = 50000
NB = 20000
EMB = 32
HEMB = EMB // 2
E_UI = 800000
E_BI = 640000
BATCH = 4096

NC, NS = 2, 16
ND = 51200
NBP = 20480
NUP = 51200
NBA = 20480
CD = 5000
CS = 2000
CB = 2000
ZR = 640

f32 = jnp.float32
i32 = jnp.int32

_MESH = plsc.VectorSubcoreMesh(core_axis_name="c", subcore_axis_name="s")
_SC_PARAMS = pltpu.CompilerParams(use_tc_tiling_on_sc=False)


def _fill_zeros(zb, width):
    def fill(i, carry):
        for w in range(width // 16):
            zb[i, pl.ds(w * 16, 16)] = jnp.zeros((16,), f32)
        return carry

    lax.fori_loop(0, ZR, fill, 0)


def _gs_serial(src_idx, table, dst_idx, acc, base, n, ck, bufs):
    sv, dv, rv, gs, ss = bufs

    def body(i, carry):
        off = base + i * ck
        pltpu.sync_copy(src_idx.at[pl.ds(off, ck)], sv)
        pltpu.sync_copy(dst_idx.at[pl.ds(off, ck)], dv)
        pltpu.async_copy(table.at[sv], rv, gs).wait()
        pltpu.sync_copy(rv, acc.at[dv], add=True)
        return carry

    lax.fori_loop(0, n, body, 0)


def _gs_pipeline(src_idx, table, dst_idx, acc, base, n, ck, b0, b1):
    def lg(c, bufs):
        sv, dv, rv, gs, _ = bufs
        off = base + c * ck
        pltpu.sync_copy(src_idx.at[pl.ds(off, ck)], sv)
        pltpu.sync_copy(dst_idx.at[pl.ds(off, ck)], dv)
        pltpu.async_copy(table.at[sv], rv, gs)

    lg(0, b0)

    def pair(i2, carry):
        for b, bufs, nbufs in ((0, b0, b1), (1, b1, b0)):
            sv, dv, rv, gs, ss = bufs
            c = i2 * 2 + b
            pltpu.make_async_copy(table.at[sv], rv, gs).wait()
            pltpu.async_copy(rv, acc.at[dv], ss, add=True)
            svn, dvn, rvn, gsn, ssn = nbufs
            if b == 0:
                @pl.when(i2 > 0)
                def _():
                    pltpu.make_async_copy(rvn, acc.at[dvn], ssn).wait()

                lg(c + 1, nbufs)
            else:
                @pl.when(i2 < n // 2 - 1)
                def _():
                    pltpu.make_async_copy(rvn, acc.at[dvn], ssn).wait()
                    lg(c + 1, nbufs)
        return carry

    lax.fori_loop(0, n // 2, pair, 0)
    for bufs in (b0, b1):
        sv, dv, rv, gs, ss = bufs
        pltpu.make_async_copy(rv, acc.at[dv], ss).wait()


def _ones_pipeline(idx_hbm, acc, ones_v, base, n, ck, i0, s0, i1, s1):
    def body(i, carry):
        pltpu.sync_copy(idx_hbm.at[pl.ds(base + i * ck, ck)], i0)
        pltpu.sync_copy(ones_v, acc.at[i0], add=True)
        return carry

    lax.fori_loop(0, n, body, 0)


def _deg_body(ui_u, ui_i, bi_b, ones_hbm,
              deg_u, deg_i, bsz,
              i0, i1, ones_v, zbuf, s0, s1, acc_deg, acc_bs):
    core = lax.axis_index("c")
    sub = lax.axis_index("s")
    dpt = ND // NS
    bpt = NBP // NS
    d0 = sub * dpt
    b0 = sub * bpt

    def fill(i, carry):
        zbuf[pl.ds(i * 16, 16)] = jnp.zeros((16,), f32)
        return carry

    lax.fori_loop(0, dpt // 16, fill, 0)
    pltpu.sync_copy(zbuf, acc_deg.at[pl.ds(d0, dpt)])
    pltpu.sync_copy(zbuf.at[pl.ds(0, bpt)], acc_bs.at[pl.ds(b0, bpt)])
    pltpu.sync_copy(ones_hbm, ones_v)
    plsc.subcore_barrier()

    @pl.when(core == 0)
    def _():
        _ones_pipeline(ui_u, acc_deg, ones_v, sub * (E_UI // NS),
                       (E_UI // NS) // CD, CD, i0, s0, i1, s1)
        _ones_pipeline(bi_b, acc_bs, ones_v, sub * (E_BI // NS),
                       (E_BI // NS) // CD, CD, i0, s0, i1, s1)

    @pl.when(core == 1)
    def _():
        _ones_pipeline(ui_i, acc_deg, ones_v, sub * (E_UI // NS),
                       (E_UI // NS) // CD, CD, i0, s0, i1, s1)

    plsc.subcore_barrier()

    @pl.when(core == 0)
    def _():
        pltpu.sync_copy(acc_deg.at[pl.ds(d0, dpt)], zbuf)
        pltpu.sync_copy(zbuf, deg_u.at[pl.ds(d0, dpt)])
        pltpu.sync_copy(acc_bs.at[pl.ds(b0, bpt)], zbuf.at[pl.ds(0, bpt)])
        pltpu.sync_copy(zbuf.at[pl.ds(0, bpt)], bsz.at[pl.ds(b0, bpt)])

    @pl.when(core == 1)
    def _():
        pltpu.sync_copy(acc_deg.at[pl.ds(d0, dpt)], zbuf)
        pltpu.sync_copy(zbuf, deg_i.at[pl.ds(d0, dpt)])


_deg_call = functools.partial(
    pl.kernel,
    out_type=(
        jax.ShapeDtypeStruct((ND,), f32),
        jax.ShapeDtypeStruct((ND,), f32),
        jax.ShapeDtypeStruct((NBP,), f32),
    ),
    mesh=_MESH,
    compiler_params=_SC_PARAMS,
    scratch_types=[
        pltpu.VMEM((CD,), i32),
        pltpu.VMEM((CD,), i32),
        pltpu.VMEM((CD,), f32),
        pltpu.VMEM((ND // NS,), f32),
        pltpu.SemaphoreType.DMA,
        pltpu.SemaphoreType.DMA,
        pltpu.VMEM_SHARED((ND,), f32),
        pltpu.VMEM_SHARED((NBP,), f32),
    ],
)(_deg_body)


def _spmm_body(gu0, gi0, gu1, gi1, ui_u, ui_i,
               hu0, hi0, hu1, hi1,
               sv, dv, rv, zb, gs, ss, acc):
    core = lax.axis_index("c")
    sub = lax.axis_index("s")
    rpt = NUP // NS
    r0 = sub * rpt
    _fill_zeros(zb, HEMB)

    def zero_acc():
        for k in range(rpt // ZR):
            pltpu.sync_copy(zb, acc.at[pl.ds(r0 + k * ZR, ZR)])

    def drain(h):
        for k in range(rpt // ZR):
            pltpu.sync_copy(acc.at[pl.ds(r0 + k * ZR, ZR)], zb)
            pltpu.sync_copy(zb, h.at[pl.ds(r0 + k * ZR, ZR)])
        _fill_zeros(zb, HEMB)
        zero_acc()

    ept = E_UI // NS
    base = sub * ept
    bufs = (sv, dv, rv, gs, ss)

    zero_acc()
    plsc.subcore_barrier()

    @pl.when(core == 0)
    def _():
        _gs_serial(ui_i, gi0, ui_u, acc, base, ept // CS, CS, bufs)

    @pl.when(core == 1)
    def _():
        _gs_serial(ui_u, gu0, ui_i, acc, base, ept // CS, CS, bufs)

    plsc.subcore_barrier()

    @pl.when(core == 0)
    def _():
        drain(hu0)

    @pl.when(core == 1)
    def _():
        drain(hi0)

    plsc.subcore_barrier()

    @pl.when(core == 0)
    def _():
        _gs_serial(ui_i, gi1, ui_u, acc, base, ept // CS, CS, bufs)

    @pl.when(core == 1)
    def _():
        _gs_serial(ui_u, gu1, ui_i, acc, base, ept // CS, CS, bufs)

    plsc.subcore_barrier()

    @pl.when(core == 0)
    def _():
        drain(hu1)

    @pl.when(core == 1)
    def _():
        drain(hi1)


_spmm_call = functools.partial(
    pl.kernel,
    out_type=(
        jax.ShapeDtypeStruct((NUP, HEMB), f32),
        jax.ShapeDtypeStruct((NUP, HEMB), f32),
        jax.ShapeDtypeStruct((NUP, HEMB), f32),
        jax.ShapeDtypeStruct((NUP, HEMB), f32),
    ),
    mesh=_MESH,
    compiler_params=_SC_PARAMS,
    scratch_types=[
        pltpu.VMEM((CS,), i32),
        pltpu.VMEM((CS,), i32),
        pltpu.VMEM((CS, HEMB), f32),
        pltpu.VMEM((ZR, HEMB), f32),
        pltpu.SemaphoreType.DMA,
        pltpu.SemaphoreType.DMA,
        pltpu.VMEM_SHARED((NUP, HEMB), f32),
    ],
)(_spmm_body)


def _bi_body(ai, bi_b, bi_i,
             hb,
             sv0, dv0, rv0, sv1, dv1, rv1, zb, gs0, ss0, gs1, ss1, acc):
    core = lax.axis_index("c")
    sub = lax.axis_index("s")
    rpt = NBA // NS
    r0 = sub * rpt
    _fill_zeros(zb, EMB)
    for k in range(rpt // ZR):
        pltpu.sync_copy(zb, acc.at[pl.ds(r0 + k * ZR, ZR)])
    plsc.subcore_barrier()

    ept = E_BI // (NC * NS)
    base = (core * NS + sub) * ept
    bufs0 = (sv0, dv0, rv0, gs0, ss0)
    bufs1 = (sv1, dv1, rv1, gs1, ss1)
    _gs_serial(bi_i, ai, bi_b, acc, base, ept // CB, CB, bufs0)
    plsc.subcore_barrier()
    for k in range(rpt // ZR):
        pltpu.sync_copy(acc.at[pl.ds(r0 + k * ZR, ZR)], zb)
        pltpu.sync_copy(zb, hb.at[pl.ds(core * NBA + r0 + k * ZR, ZR)])


_bi_call = functools.partial(
    pl.kernel,
    out_type=jax.ShapeDtypeStruct((NC * NBA, EMB), f32),
    mesh=_MESH,
    compiler_params=_SC_PARAMS,
    scratch_types=[
        pltpu.VMEM((CB,), i32),
        pltpu.VMEM((CB,), i32),
        pltpu.VMEM((CB, EMB), f32),
        pltpu.VMEM((8,), i32),
        pltpu.VMEM((8,), i32),
        pltpu.VMEM((8, EMB), f32),
        pltpu.VMEM((ZR, EMB), f32),
        pltpu.SemaphoreType.DMA,
        pltpu.SemaphoreType.DMA,
        pltpu.SemaphoreType.DMA,
        pltpu.SemaphoreType.DMA,
        pltpu.VMEM_SHARED((NBA, EMB), f32),
    ],
)(_bi_body)


UPW = BATCH // (NC * NS)
BPW = 2 * BATCH // (NC * NS)


def _lookup_body(au, brep, uidx, bidx,
                 ue, be,
                 iu_v, ib_v, ru_v, rb_v, sem):
    core = lax.axis_index("c")
    sub = lax.axis_index("s")
    wid = core * NS + sub
    u0 = wid * UPW
    pltpu.sync_copy(uidx.at[pl.ds(u0, UPW)], iu_v)
    pltpu.async_copy(au.at[iu_v], ru_v, sem).wait()
    pltpu.sync_copy(ru_v, ue.at[pl.ds(u0, UPW)])
    b0 = wid * BPW
    pltpu.sync_copy(bidx.at[pl.ds(b0, BPW)], ib_v)
    pltpu.async_copy(brep.at[ib_v], rb_v, sem).wait()
    pltpu.sync_copy(rb_v, be.at[pl.ds(b0, BPW)])


_lookup_call = functools.partial(
    pl.kernel,
    out_type=(
        jax.ShapeDtypeStruct((BATCH, EMB), f32),
        jax.ShapeDtypeStruct((2 * BATCH, EMB), f32),
    ),
    mesh=_MESH,
    compiler_params=_SC_PARAMS,
    scratch_types=[
        pltpu.VMEM((UPW,), i32),
        pltpu.VMEM((BPW,), i32),
        pltpu.VMEM((UPW, EMB), f32),
        pltpu.VMEM((BPW, EMB), f32),
        pltpu.SemaphoreType.DMA,
    ],
)(_lookup_body)


def _loss_body(u_ref, pos_ref, neg_ref, out_ref):
    u = u_ref[...]
    x = jnp.sum(u * (neg_ref[...] - pos_ref[...]), axis=1)
    sp = jnp.maximum(x, 0.0) + jnp.log(1.0 + jnp.exp(-jnp.abs(x)))
    out_ref[...] = jnp.broadcast_to(jnp.mean(sp), (1, 1))


def _loss_call(ue, pos, neg):
    return pl.pallas_call(
        _loss_body,
        out_shape=jax.ShapeDtypeStruct((1, 1), f32),
    )(ue, pos, neg)


def kernel(users_feature, items_feature, bundles_feature,
           ui_u, ui_i, bi_b, bi_i, users, bundles):
    ui_u = ui_u.astype(i32)
    ui_i = ui_i.astype(i32)
    bi_b = bi_b.astype(i32)
    bi_i = bi_i.astype(i32)

    ones_c = jnp.ones((CD,), f32)

    deg_u, deg_i, bsz = _deg_call(ui_u, ui_i, bi_b, ones_c)
    ru = 1.0 / (jnp.sqrt(deg_u[:NU]) + 1e-8)
    ri = 1.0 / (jnp.sqrt(deg_i[:NI]) + 1e-8)
    binv = 1.0 / (bsz[:NB] + 1e-8)

    def spmm(gu, gi):
        hu0, hi0, hu1, hi1 = _spmm_call(
            gu[:, :HEMB], gi[:, :HEMB], gu[:, HEMB:], gi[:, HEMB:],
            ui_u, ui_i)
        hu = jnp.concatenate([hu0[:NU], hu1[:NU]], axis=1)
        hi = jnp.concatenate([hi0[:NI], hi1[:NI]], axis=1)
        return hu, hi

    g0u = users_feature * ru[:, None]
    g0i = items_feature * ri[:, None]
    h1u, h1i = spmm(g0u, g0i)
    f1u = h1u * ru[:, None]
    f1i = h1i * ri[:, None]
    h2u, h2i = spmm(f1u * ru[:, None], f1i * ri[:, None])
    f2u = h2u * ru[:, None]
    f2i = h2i * ri[:, None]

    au = (users_feature + f1u + f2u) / 3.0
    ai = (items_feature + f1i + f2i) / 3.0

    hb = _bi_call(ai, bi_b, bi_i)
    brep = (hb[:NB] + hb[NBA:NBA + NB]) * binv[:, None]

    ue, be = _lookup_call(au, brep,
                          users.reshape(-1).astype(i32),
                          bundles.reshape(-1).astype(i32))
    be = be.reshape(BATCH, 2, EMB)
    loss = _loss_call(ue, be[:, 0, :], be[:, 1, :])
    return (loss[0, 0], jnp.zeros(1, f32))


# speedup vs baseline: 1.3364x; 1.3364x over previous
import functools

import jax
import jax.numpy as jnp
from jax import lax
from jax.experimental import pallas as pl
from jax.experimental.pallas import tpu as pltpu
from jax.experimental.pallas import tpu_sc as plsc

NU = 50000
NI = 50000
NB = 20000
EMB = 32
HEMB = EMB // 2
E_UI = 800000
E_BI = 640000
BATCH = 4096

NC, NS = 2, 16
ND = 51200
NBP = 20480
NUP = 50048
ZRS = 184
NBA = 20480
CD = 5000
CS = 2000
CB = 2000
ZR = 640

f32 = jnp.float32
i32 = jnp.int32

_MESH = plsc.VectorSubcoreMesh(core_axis_name="c", subcore_axis_name="s")
_SC_PARAMS = pltpu.CompilerParams(use_tc_tiling_on_sc=False)


def _fill_zeros(zb, rows, width):
    def fill(i, carry):
        for w in range(width // 16):
            zb[i, pl.ds(w * 16, 16)] = jnp.zeros((16,), f32)
        return carry

    lax.fori_loop(0, rows, fill, 0)


def _gs_serial(src_idx, table, dst_idx, acc, base, n, ck, bufs):
    sv, dv, rv, gs, ss = bufs

    def body(i, carry):
        off = base + i * ck
        pltpu.sync_copy(src_idx.at[pl.ds(off, ck)], sv)
        pltpu.sync_copy(dst_idx.at[pl.ds(off, ck)], dv)
        pltpu.async_copy(table.at[sv], rv, gs).wait()
        pltpu.sync_copy(rv, acc.at[dv], add=True)
        return carry

    lax.fori_loop(0, n, body, 0)


def _gs_pipeline(src_idx, table, dst_idx, acc, base, n, ck, b0, b1):
    def lg(c, bufs):
        sv, dv, rv, gs, _ = bufs
        off = base + c * ck
        pltpu.sync_copy(src_idx.at[pl.ds(off, ck)], sv)
        pltpu.sync_copy(dst_idx.at[pl.ds(off, ck)], dv)
        pltpu.async_copy(table.at[sv], rv, gs)

    lg(0, b0)

    def pair(i2, carry):
        for b, bufs, nbufs in ((0, b0, b1), (1, b1, b0)):
            sv, dv, rv, gs, ss = bufs
            c = i2 * 2 + b
            pltpu.make_async_copy(table.at[sv], rv, gs).wait()
            pltpu.async_copy(rv, acc.at[dv], ss, add=True)
            svn, dvn, rvn, gsn, ssn = nbufs
            if b == 0:
                @pl.when(i2 > 0)
                def _():
                    pltpu.make_async_copy(rvn, acc.at[dvn], ssn).wait()

                lg(c + 1, nbufs)
            else:
                @pl.when(i2 < n // 2 - 1)
                def _():
                    pltpu.make_async_copy(rvn, acc.at[dvn], ssn).wait()
                    lg(c + 1, nbufs)
        return carry

    lax.fori_loop(0, n // 2, pair, 0)
    for bufs in (b0, b1):
        sv, dv, rv, gs, ss = bufs
        pltpu.make_async_copy(rv, acc.at[dv], ss).wait()


def _gs_full(src_idx, table, dst_idx, acc, base, n, ck, bufs):
    sv, dv, rv, gs, ss = bufs

    def body(i, carry):
        off = base + i * ck
        pltpu.sync_copy(src_idx.at[pl.ds(off, ck)], sv)
        pltpu.sync_copy(dst_idx.at[pl.ds(off, ck)], dv)
        pltpu.async_copy(table.at[sv], rv, gs).wait()
        pltpu.sync_copy(rv, acc.at[dv], add=True)
        return carry

    lax.fori_loop(0, n, body, 0)


def _gs_pipe(src_idx, table, dst_idx, acc, base, n, ck, b0, b1):
    def lg(c, bufs):
        sv, dv, rv, gs, _ = bufs
        off = base + c * ck
        pltpu.sync_copy(src_idx.at[pl.ds(off, ck)], sv)
        pltpu.sync_copy(dst_idx.at[pl.ds(off, ck)], dv)
        pltpu.async_copy(table.at[sv], rv, gs)

    def gwait(bufs):
        sv, dv, rv, gs, _ = bufs
        pltpu.make_async_copy(table.at[sv], rv, gs).wait()

    def sc_start(bufs):
        _, dv, rv, _, ss = bufs
        pltpu.async_copy(rv, acc.at[dv], ss, add=True)

    def sc_wait(bufs):
        _, dv, rv, _, ss = bufs
        pltpu.make_async_copy(rv, acc.at[dv], ss).wait()

    lg(0, b0)
    gwait(b0)
    sc_start(b0)
    lg(1, b1)

    def pair(i2, carry):
        gwait(b1)
        sc_start(b1)
        sc_wait(b0)
        lg(2 * i2 + 2, b0)
        gwait(b0)
        sc_start(b0)
        sc_wait(b1)

        @pl.when(i2 < (n - 1) // 2 - 1)
        def _():
            lg(2 * i2 + 3, b1)

        return carry

    lax.fori_loop(0, (n - 1) // 2, pair, 0)
    sc_wait(b0)


def _ones_pipeline(idx_hbm, acc, ones_v, base, n, ck, i0, s0, i1, s1):
    def body(i, carry):
        pltpu.sync_copy(idx_hbm.at[pl.ds(base + i * ck, ck)], i0)
        pltpu.sync_copy(ones_v, acc.at[i0], add=True)
        return carry

    lax.fori_loop(0, n, body, 0)


def _deg_body(ui_u, ui_i, bi_b, ones_hbm,
              deg_u, deg_i, bsz,
              i0, i1, ones_v, zbuf, s0, s1, acc_deg, acc_bs):
    core = lax.axis_index("c")
    sub = lax.axis_index("s")
    dpt = ND // NS
    bpt = NBP // NS
    d0 = sub * dpt
    b0 = sub * bpt

    def fill(i, carry):
        zbuf[pl.ds(i * 16, 16)] = jnp.zeros((16,), f32)
        return carry

    lax.fori_loop(0, dpt // 16, fill, 0)
    pltpu.sync_copy(zbuf, acc_deg.at[pl.ds(d0, dpt)])
    pltpu.sync_copy(zbuf.at[pl.ds(0, bpt)], acc_bs.at[pl.ds(b0, bpt)])
    pltpu.sync_copy(ones_hbm, ones_v)
    plsc.subcore_barrier()

    @pl.when(core == 0)
    def _():
        _ones_pipeline(ui_u, acc_deg, ones_v, sub * (E_UI // NS),
                       (E_UI // NS) // CD, CD, i0, s0, i1, s1)
        _ones_pipeline(bi_b, acc_bs, ones_v, sub * (E_BI // NS),
                       (E_BI // NS) // CD, CD, i0, s0, i1, s1)

    @pl.when(core == 1)
    def _():
        _ones_pipeline(ui_i, acc_deg, ones_v, sub * (E_UI // NS),
                       (E_UI // NS) // CD, CD, i0, s0, i1, s1)

    plsc.subcore_barrier()

    @pl.when(core == 0)
    def _():
        pltpu.sync_copy(acc_deg.at[pl.ds(d0, dpt)], zbuf)
        pltpu.sync_copy(zbuf, deg_u.at[pl.ds(d0, dpt)])
        pltpu.sync_copy(acc_bs.at[pl.ds(b0, bpt)], zbuf.at[pl.ds(0, bpt)])
        pltpu.sync_copy(zbuf.at[pl.ds(0, bpt)], bsz.at[pl.ds(b0, bpt)])

    @pl.when(core == 1)
    def _():
        pltpu.sync_copy(acc_deg.at[pl.ds(d0, dpt)], zbuf)
        pltpu.sync_copy(zbuf, deg_i.at[pl.ds(d0, dpt)])


_deg_call = functools.partial(
    pl.kernel,
    out_type=(
        jax.ShapeDtypeStruct((ND,), f32),
        jax.ShapeDtypeStruct((ND,), f32),
        jax.ShapeDtypeStruct((NBP,), f32),
    ),
    mesh=_MESH,
    compiler_params=_SC_PARAMS,
    scratch_types=[
        pltpu.VMEM((CD,), i32),
        pltpu.VMEM((CD,), i32),
        pltpu.VMEM((CD,), f32),
        pltpu.VMEM((ND // NS,), f32),
        pltpu.SemaphoreType.DMA,
        pltpu.SemaphoreType.DMA,
        pltpu.VMEM_SHARED((ND,), f32),
        pltpu.VMEM_SHARED((NBP,), f32),
    ],
)(_deg_body)


def _spmm_body(gu0, gi0, gu1, gi1, ui_u, ui_i,
               hu, hi,
               sv, dv, rv, sv2, dv2, rv2, zb, gs, ss, gs2, ss2, acc):
    core = lax.axis_index("c")
    sub = lax.axis_index("s")
    rpt = NUP // NS
    r0 = sub * rpt
    _fill_zeros(zb, ZRS, HEMB)

    def zero_acc():
        for k in range(rpt // ZRS):
            pltpu.sync_copy(zb, acc.at[pl.ds(r0 + k * ZRS, ZRS)])

    def drain(h, wcol):
        for k in range(rpt // ZRS):
            pltpu.sync_copy(acc.at[pl.ds(r0 + k * ZRS, ZRS)], zb)
            pltpu.sync_copy(
                zb, h.at[pl.ds(r0 + k * ZRS, ZRS), pl.ds(wcol, HEMB)])

    ept = E_UI // NS
    base = sub * ept
    bufs = (sv, dv, rv, gs, ss)
    bufsB = (sv2, dv2, rv2, gs2, ss2)

    for w, (guw, giw) in enumerate(((gu0, gi0), (gu1, gi1))):
        wcol = w * HEMB
        zero_acc()
        plsc.subcore_barrier()

        @pl.when(core == 0)
        def _():
            _gs_pipe(ui_i, giw, ui_u, acc, base, ept // CS, CS, bufs, bufsB)

        @pl.when(core == 1)
        def _():
            _gs_pipe(ui_u, guw, ui_i, acc, base, ept // CS, CS, bufs, bufsB)

        plsc.subcore_barrier()

        @pl.when(core == 0)
        def _():
            drain(hu, wcol)

        @pl.when(core == 1)
        def _():
            drain(hi, wcol)

        plsc.subcore_barrier()


_spmm_call = functools.partial(
    pl.kernel,
    out_type=(
        jax.ShapeDtypeStruct((NUP, EMB), f32),
        jax.ShapeDtypeStruct((NUP, EMB), f32),
    ),
    mesh=_MESH,
    compiler_params=_SC_PARAMS,
    scratch_types=[
        pltpu.VMEM((CS,), i32),
        pltpu.VMEM((CS,), i32),
        pltpu.VMEM((CS, HEMB), f32),
        pltpu.VMEM((CS,), i32),
        pltpu.VMEM((CS,), i32),
        pltpu.VMEM((CS, HEMB), f32),
        pltpu.VMEM((ZRS, HEMB), f32),
        pltpu.SemaphoreType.DMA,
        pltpu.SemaphoreType.DMA,
        pltpu.SemaphoreType.DMA,
        pltpu.SemaphoreType.DMA,
        pltpu.VMEM_SHARED((NUP, HEMB), f32),
    ],
)(_spmm_body)


def _bi_body(ai, bi_b, bi_i,
             hb,
             sv0, dv0, rv0, sv1, dv1, rv1, zb, gs0, ss0, gs1, ss1, acc):
    core = lax.axis_index("c")
    sub = lax.axis_index("s")
    rpt = NBA // NS
    r0 = sub * rpt
    _fill_zeros(zb, ZR, EMB)
    for k in range(rpt // ZR):
        pltpu.sync_copy(zb, acc.at[pl.ds(r0 + k * ZR, ZR)])
    plsc.subcore_barrier()

    ept = E_BI // (NC * NS)
    base = (core * NS + sub) * ept
    bufs0 = (sv0, dv0, rv0, gs0, ss0)
    bufs1 = (sv1, dv1, rv1, gs1, ss1)
    _gs_full(bi_i, ai, bi_b, acc, base, ept // CB, CB, bufs0)
    plsc.subcore_barrier()
    for k in range(rpt // ZR):
        pltpu.sync_copy(acc.at[pl.ds(r0 + k * ZR, ZR)], zb)
        pltpu.sync_copy(zb, hb.at[pl.ds(core * NBA + r0 + k * ZR, ZR)])


_bi_call = functools.partial(
    pl.kernel,
    out_type=jax.ShapeDtypeStruct((NC * NBA, EMB), f32),
    mesh=_MESH,
    compiler_params=_SC_PARAMS,
    scratch_types=[
        pltpu.VMEM((CB,), i32),
        pltpu.VMEM((CB,), i32),
        pltpu.VMEM((CB, EMB), f32),
        pltpu.VMEM((8,), i32),
        pltpu.VMEM((8,), i32),
        pltpu.VMEM((8, EMB), f32),
        pltpu.VMEM((ZR, EMB), f32),
        pltpu.SemaphoreType.DMA,
        pltpu.SemaphoreType.DMA,
        pltpu.SemaphoreType.DMA,
        pltpu.SemaphoreType.DMA,
        pltpu.VMEM_SHARED((NBA, EMB), f32),
    ],
)(_bi_body)


UPW = BATCH // (NC * NS)
BPW = 2 * BATCH // (NC * NS)


def _lookup_body(au, brep, uidx, bidx,
                 ue, be,
                 iu_v, ib_v, ru_v, rb_v, sem):
    core = lax.axis_index("c")
    sub = lax.axis_index("s")
    wid = core * NS + sub
    u0 = wid * UPW
    pltpu.sync_copy(uidx.at[pl.ds(u0, UPW)], iu_v)
    pltpu.async_copy(au.at[iu_v], ru_v, sem).wait()
    pltpu.sync_copy(ru_v, ue.at[pl.ds(u0, UPW)])
    b0 = wid * BPW
    pltpu.sync_copy(bidx.at[pl.ds(b0, BPW)], ib_v)
    pltpu.async_copy(brep.at[ib_v], rb_v, sem).wait()
    pltpu.sync_copy(rb_v, be.at[pl.ds(b0, BPW)])


_lookup_call = functools.partial(
    pl.kernel,
    out_type=(
        jax.ShapeDtypeStruct((BATCH, EMB), f32),
        jax.ShapeDtypeStruct((2 * BATCH, EMB), f32),
    ),
    mesh=_MESH,
    compiler_params=_SC_PARAMS,
    scratch_types=[
        pltpu.VMEM((UPW,), i32),
        pltpu.VMEM((BPW,), i32),
        pltpu.VMEM((UPW, EMB), f32),
        pltpu.VMEM((BPW, EMB), f32),
        pltpu.SemaphoreType.DMA,
    ],
)(_lookup_body)


def _loss_body(u_ref, pos_ref, neg_ref, out_ref):
    u = u_ref[...]
    x = jnp.sum(u * (neg_ref[...] - pos_ref[...]), axis=1)
    sp = jnp.maximum(x, 0.0) + jnp.log(1.0 + jnp.exp(-jnp.abs(x)))
    out_ref[...] = jnp.broadcast_to(jnp.mean(sp), (1, 1))


def _loss_call(ue, pos, neg):
    return pl.pallas_call(
        _loss_body,
        out_shape=jax.ShapeDtypeStruct((1, 1), f32),
    )(ue, pos, neg)


def kernel(users_feature, items_feature, bundles_feature,
           ui_u, ui_i, bi_b, bi_i, users, bundles):
    ui_u = ui_u.astype(i32)
    ui_i = ui_i.astype(i32)
    bi_b = bi_b.astype(i32)
    bi_i = bi_i.astype(i32)

    ones_c = jnp.ones((CD,), f32)

    deg_u, deg_i, bsz = _deg_call(ui_u, ui_i, bi_b, ones_c)
    ru = 1.0 / (jnp.sqrt(deg_u[:NU]) + 1e-8)
    ri = 1.0 / (jnp.sqrt(deg_i[:NI]) + 1e-8)
    binv = 1.0 / (bsz[:NB] + 1e-8)

    def spmm(su, si, fu, fi):
        hu, hi = _spmm_call(
            fu[:, :HEMB] * su[:, None], fi[:, :HEMB] * si[:, None],
            fu[:, HEMB:] * su[:, None], fi[:, HEMB:] * si[:, None],
            ui_u, ui_i)
        return hu[:NU], hi[:NI]

    h1u, h1i = spmm(ru, ri, users_feature, items_feature)
    h2u, h2i = spmm(ru * ru, ri * ri, h1u, h1i)

    au = (users_feature + (h1u + h2u) * ru[:, None]) / 3.0
    ai = (items_feature + (h1i + h2i) * ri[:, None]) / 3.0

    hb = _bi_call(ai, bi_b, bi_i)
    brep = (hb[:NB] + hb[NBA:NBA + NB]) * binv[:, None]

    ue, be = _lookup_call(au, brep,
                          users.reshape(-1).astype(i32),
                          bundles.reshape(-1).astype(i32))
    be = be.reshape(BATCH, 2, EMB)
    loss = _loss_call(ue, be[:, 0, :], be[:, 1, :])
    return (loss[0, 0], jnp.zeros(1, f32))


# speedup vs baseline: 1.5232x; 1.1398x over previous
import functools

import jax
import jax.numpy as jnp
from jax import lax
from jax.experimental import pallas as pl
from jax.experimental.pallas import tpu as pltpu
from jax.experimental.pallas import tpu_sc as plsc

NU = 50000
NI = 50000
NB = 20000
EMB = 32
HEMB = EMB // 2
E_UI = 800000
E_BI = 640000
BATCH = 4096

NC, NS = 2, 16
ND = 51200
NBP = 20480
NUP = 50048
ZRS = 184
NBA = 20480
CD = 5000
CS = 2000
CB = 1000
ZR = 640

f32 = jnp.float32
i32 = jnp.int32

_MESH = plsc.VectorSubcoreMesh(core_axis_name="c", subcore_axis_name="s")
_SC_PARAMS = pltpu.CompilerParams(use_tc_tiling_on_sc=False)


def _fill_zeros(zb, rows, width):
    def fill(i, carry):
        for w in range(width // 16):
            zb[i, pl.ds(w * 16, 16)] = jnp.zeros((16,), f32)
        return carry

    lax.fori_loop(0, rows, fill, 0)


def _gs_serial(src_idx, table, dst_idx, acc, base, n, ck, bufs):
    sv, dv, rv, gs, ss = bufs

    def body(i, carry):
        off = base + i * ck
        pltpu.sync_copy(src_idx.at[pl.ds(off, ck)], sv)
        pltpu.sync_copy(dst_idx.at[pl.ds(off, ck)], dv)
        pltpu.async_copy(table.at[sv], rv, gs).wait()
        pltpu.sync_copy(rv, acc.at[dv], add=True)
        return carry

    lax.fori_loop(0, n, body, 0)


def _gs_pipeline(src_idx, table, dst_idx, acc, base, n, ck, b0, b1):
    def lg(c, bufs):
        sv, dv, rv, gs, _ = bufs
        off = base + c * ck
        pltpu.sync_copy(src_idx.at[pl.ds(off, ck)], sv)
        pltpu.sync_copy(dst_idx.at[pl.ds(off, ck)], dv)
        pltpu.async_copy(table.at[sv], rv, gs)

    lg(0, b0)

    def pair(i2, carry):
        for b, bufs, nbufs in ((0, b0, b1), (1, b1, b0)):
            sv, dv, rv, gs, ss = bufs
            c = i2 * 2 + b
            pltpu.make_async_copy(table.at[sv], rv, gs).wait()
            pltpu.async_copy(rv, acc.at[dv], ss, add=True)
            svn, dvn, rvn, gsn, ssn = nbufs
            if b == 0:
                @pl.when(i2 > 0)
                def _():
                    pltpu.make_async_copy(rvn, acc.at[dvn], ssn).wait()

                lg(c + 1, nbufs)
            else:
                @pl.when(i2 < n // 2 - 1)
                def _():
                    pltpu.make_async_copy(rvn, acc.at[dvn], ssn).wait()
                    lg(c + 1, nbufs)
        return carry

    lax.fori_loop(0, n // 2, pair, 0)
    for bufs in (b0, b1):
        sv, dv, rv, gs, ss = bufs
        pltpu.make_async_copy(rv, acc.at[dv], ss).wait()


def _gs_full(src_idx, table, dst_idx, acc, base, n, ck, bufs):
    sv, dv, rv, gs, ss = bufs

    def body(i, carry):
        off = base + i * ck
        pltpu.sync_copy(src_idx.at[pl.ds(off, ck)], sv)
        pltpu.sync_copy(dst_idx.at[pl.ds(off, ck)], dv)
        pltpu.async_copy(table.at[sv], rv, gs).wait()
        pltpu.sync_copy(rv, acc.at[dv], add=True)
        return carry

    lax.fori_loop(0, n, body, 0)


def _gs_pipe(src_idx, table, dst_idx, acc, base, n, ck, b0, b1):
    def lg(c, bufs):
        sv, dv, rv, gs, _ = bufs
        off = base + c * ck
        pltpu.sync_copy(src_idx.at[pl.ds(off, ck)], sv)
        pltpu.sync_copy(dst_idx.at[pl.ds(off, ck)], dv)
        pltpu.async_copy(table.at[sv], rv, gs)

    def gwait(bufs):
        sv, dv, rv, gs, _ = bufs
        pltpu.make_async_copy(table.at[sv], rv, gs).wait()

    def sc_start(bufs):
        _, dv, rv, _, ss = bufs
        pltpu.async_copy(rv, acc.at[dv], ss, add=True)

    def sc_wait(bufs):
        _, dv, rv, _, ss = bufs
        pltpu.make_async_copy(rv, acc.at[dv], ss).wait()

    lg(0, b0)
    gwait(b0)
    sc_start(b0)
    lg(1, b1)

    def pair(i2, carry):
        gwait(b1)
        sc_start(b1)
        sc_wait(b0)
        lg(2 * i2 + 2, b0)
        gwait(b0)
        sc_start(b0)
        sc_wait(b1)

        @pl.when(i2 < (n - 1) // 2 - 1)
        def _():
            lg(2 * i2 + 3, b1)

        return carry

    lax.fori_loop(0, (n - 1) // 2, pair, 0)
    sc_wait(b0)


def _ones_pipeline(idx_hbm, acc, ones_v, base, n, ck, i0, s0, i1, s1):
    def body(i, carry):
        pltpu.sync_copy(idx_hbm.at[pl.ds(base + i * ck, ck)], i0)
        pltpu.sync_copy(ones_v, acc.at[i0], add=True)
        return carry

    lax.fori_loop(0, n, body, 0)


def _deg_body(ui_u, ui_i, bi_b, ones_hbm,
              deg_u, deg_i, bsz,
              i0, i1, ones_v, zbuf, s0, s1, acc_deg, acc_bs):
    core = lax.axis_index("c")
    sub = lax.axis_index("s")
    dpt = ND // NS
    bpt = NBP // NS
    d0 = sub * dpt
    b0 = sub * bpt

    def fill(i, carry):
        zbuf[pl.ds(i * 16, 16)] = jnp.zeros((16,), f32)
        return carry

    lax.fori_loop(0, dpt // 16, fill, 0)
    pltpu.sync_copy(zbuf, acc_deg.at[pl.ds(d0, dpt)])
    pltpu.sync_copy(zbuf.at[pl.ds(0, bpt)], acc_bs.at[pl.ds(b0, bpt)])
    pltpu.sync_copy(ones_hbm, ones_v)
    plsc.subcore_barrier()

    @pl.when(core == 0)
    def _():
        _ones_pipeline(ui_u, acc_deg, ones_v, sub * (E_UI // NS),
                       (E_UI // NS) // CD, CD, i0, s0, i1, s1)
        _ones_pipeline(bi_b, acc_bs, ones_v, sub * (E_BI // NS),
                       (E_BI // NS) // CD, CD, i0, s0, i1, s1)

    @pl.when(core == 1)
    def _():
        _ones_pipeline(ui_i, acc_deg, ones_v, sub * (E_UI // NS),
                       (E_UI // NS) // CD, CD, i0, s0, i1, s1)

    plsc.subcore_barrier()

    @pl.when(core == 0)
    def _():
        pltpu.sync_copy(acc_deg.at[pl.ds(d0, dpt)], zbuf)
        pltpu.sync_copy(zbuf, deg_u.at[pl.ds(d0, dpt)])
        pltpu.sync_copy(acc_bs.at[pl.ds(b0, bpt)], zbuf.at[pl.ds(0, bpt)])
        pltpu.sync_copy(zbuf.at[pl.ds(0, bpt)], bsz.at[pl.ds(b0, bpt)])

    @pl.when(core == 1)
    def _():
        pltpu.sync_copy(acc_deg.at[pl.ds(d0, dpt)], zbuf)
        pltpu.sync_copy(zbuf, deg_i.at[pl.ds(d0, dpt)])


_deg_call = functools.partial(
    pl.kernel,
    out_type=(
        jax.ShapeDtypeStruct((ND,), f32),
        jax.ShapeDtypeStruct((ND,), f32),
        jax.ShapeDtypeStruct((NBP,), f32),
    ),
    mesh=_MESH,
    compiler_params=_SC_PARAMS,
    scratch_types=[
        pltpu.VMEM((CD,), i32),
        pltpu.VMEM((CD,), i32),
        pltpu.VMEM((CD,), f32),
        pltpu.VMEM((ND // NS,), f32),
        pltpu.SemaphoreType.DMA,
        pltpu.SemaphoreType.DMA,
        pltpu.VMEM_SHARED((ND,), f32),
        pltpu.VMEM_SHARED((NBP,), f32),
    ],
)(_deg_body)


def _make_spmm_body(scaled_out):
    def body(*args):
        if scaled_out:
            (gu0, gi0, gu1, gi1, ui_u, ui_i, su2, si2,
             hu, hi, tu0, ti0, tu1, ti1,
             sv, dv, rv, sv2, dv2, rv2, zb, s_v, gs, ss, gs2, ss2,
             acc) = args
        else:
            (gu0, gi0, gu1, gi1, ui_u, ui_i,
             hu, hi,
             sv, dv, rv, sv2, dv2, rv2, zb, gs, ss, gs2, ss2, acc) = args
        core = lax.axis_index("c")
        sub = lax.axis_index("s")
        rpt = NUP // NS
        r0 = sub * rpt
        _fill_zeros(zb, ZRS, HEMB)
        if scaled_out:
            @pl.when(core == 0)
            def _():
                pltpu.sync_copy(su2.at[pl.ds(r0, rpt)], s_v.at[pl.ds(0, rpt)])

            @pl.when(core == 1)
            def _():
                pltpu.sync_copy(si2.at[pl.ds(r0, rpt)], s_v.at[pl.ds(0, rpt)])

        def zero_acc():
            for k in range(rpt // ZRS):
                pltpu.sync_copy(zb, acc.at[pl.ds(r0 + k * ZRS, ZRS)])

        def drain(h, t, wcol):
            for k in range(rpt // ZRS):
                pltpu.sync_copy(acc.at[pl.ds(r0 + k * ZRS, ZRS)], zb)
                pltpu.sync_copy(
                    zb, h.at[pl.ds(r0 + k * ZRS, ZRS), pl.ds(wcol, HEMB)])
                if scaled_out:
                    def scale(g, carry):
                        sval = s_v[pl.ds(k * ZRS + g * 8, 16)]
                        for u in range(8):
                            row = g * 8 + u
                            zb[row, pl.ds(0, HEMB)] = (
                                zb[row, pl.ds(0, HEMB)] * sval[u])
                        return carry

                    lax.fori_loop(0, ZRS // 8, scale, 0)
                    pltpu.sync_copy(zb, t.at[pl.ds(r0 + k * ZRS, ZRS)])
            _fill_zeros(zb, ZRS, HEMB)

        ept = E_UI // NS
        base = sub * ept
        bufs = (sv, dv, rv, gs, ss)
        bufsB = (sv2, dv2, rv2, gs2, ss2)

        touts = ((tu0, ti0), (tu1, ti1)) if scaled_out else ((hu, hi),) * 2
        for w, (guw, giw) in enumerate(((gu0, gi0), (gu1, gi1))):
            wcol = w * HEMB
            zero_acc()
            plsc.subcore_barrier()

            @pl.when(core == 0)
            def _():
                _gs_pipe(ui_i, giw, ui_u, acc, base, ept // CS, CS,
                         bufs, bufsB)

            @pl.when(core == 1)
            def _():
                _gs_pipe(ui_u, guw, ui_i, acc, base, ept // CS, CS,
                         bufs, bufsB)

            plsc.subcore_barrier()
            tw = touts[w]

            @pl.when(core == 0)
            def _():
                drain(hu, tw[0], wcol)

            @pl.when(core == 1)
            def _():
                drain(hi, tw[1], wcol)

            plsc.subcore_barrier()
    return body


_SPMM_SCRATCH = [
    pltpu.VMEM((CS,), i32),
    pltpu.VMEM((CS,), i32),
    pltpu.VMEM((CS, HEMB), f32),
    pltpu.VMEM((CS,), i32),
    pltpu.VMEM((CS,), i32),
    pltpu.VMEM((CS, HEMB), f32),
    pltpu.VMEM((ZRS, HEMB), f32),
    pltpu.SemaphoreType.DMA,
    pltpu.SemaphoreType.DMA,
    pltpu.SemaphoreType.DMA,
    pltpu.SemaphoreType.DMA,
    pltpu.VMEM_SHARED((NUP, HEMB), f32),
]

_spmm_call = functools.partial(
    pl.kernel,
    out_type=(
        jax.ShapeDtypeStruct((NUP, EMB), f32),
        jax.ShapeDtypeStruct((NUP, EMB), f32),
    ),
    mesh=_MESH,
    compiler_params=_SC_PARAMS,
    scratch_types=_SPMM_SCRATCH,
)(_make_spmm_body(False))

_spmm_scaled_call = functools.partial(
    pl.kernel,
    out_type=(
        jax.ShapeDtypeStruct((NUP, EMB), f32),
        jax.ShapeDtypeStruct((NUP, EMB), f32),
        jax.ShapeDtypeStruct((NUP, HEMB), f32),
        jax.ShapeDtypeStruct((NUP, HEMB), f32),
        jax.ShapeDtypeStruct((NUP, HEMB), f32),
        jax.ShapeDtypeStruct((NUP, HEMB), f32),
    ),
    mesh=_MESH,
    compiler_params=_SC_PARAMS,
    scratch_types=(_SPMM_SCRATCH[:6]
                   + [pltpu.VMEM((ZRS, HEMB), f32),
                      pltpu.VMEM((NUP // NS + 16,), f32)]
                   + _SPMM_SCRATCH[7:]),
)(_make_spmm_body(True))


def _bi_body(ai, bi_b, bi_i,
             hb,
             sv0, dv0, rv0, sv1, dv1, rv1, zb, gs0, ss0, gs1, ss1, acc):
    core = lax.axis_index("c")
    sub = lax.axis_index("s")
    rpt = NBA // NS
    r0 = sub * rpt
    _fill_zeros(zb, ZR, EMB)
    for k in range(rpt // ZR):
        pltpu.sync_copy(zb, acc.at[pl.ds(r0 + k * ZR, ZR)])
    plsc.subcore_barrier()

    ept = E_BI // (NC * NS)
    base = (core * NS + sub) * ept
    bufs0 = (sv0, dv0, rv0, gs0, ss0)
    bufs1 = (sv1, dv1, rv1, gs1, ss1)
    nch = ept // CB
    _gs_pipe(bi_i, ai, bi_b, acc, base, nch - 1, CB, bufs0, bufs1)
    _gs_full(bi_i, ai, bi_b, acc, base + (nch - 1) * CB, 1, CB, bufs0)
    plsc.subcore_barrier()
    for k in range(rpt // ZR):
        pltpu.sync_copy(acc.at[pl.ds(r0 + k * ZR, ZR)], zb)
        pltpu.sync_copy(zb, hb.at[pl.ds(core * NBA + r0 + k * ZR, ZR)])


_bi_call = functools.partial(
    pl.kernel,
    out_type=jax.ShapeDtypeStruct((NC * NBA, EMB), f32),
    mesh=_MESH,
    compiler_params=_SC_PARAMS,
    scratch_types=[
        pltpu.VMEM((CB,), i32),
        pltpu.VMEM((CB,), i32),
        pltpu.VMEM((CB, EMB), f32),
        pltpu.VMEM((CB,), i32),
        pltpu.VMEM((CB,), i32),
        pltpu.VMEM((CB, EMB), f32),
        pltpu.VMEM((ZR, EMB), f32),
        pltpu.SemaphoreType.DMA,
        pltpu.SemaphoreType.DMA,
        pltpu.SemaphoreType.DMA,
        pltpu.SemaphoreType.DMA,
        pltpu.VMEM_SHARED((NBA, EMB), f32),
    ],
)(_bi_body)


UPW = BATCH // (NC * NS)
BPW = 2 * BATCH // (NC * NS)


def _lookup_body(au, brep, uidx, bidx,
                 ue, be,
                 iu_v, ib_v, ru_v, rb_v, sem):
    core = lax.axis_index("c")
    sub = lax.axis_index("s")
    wid = core * NS + sub
    u0 = wid * UPW
    pltpu.sync_copy(uidx.at[pl.ds(u0, UPW)], iu_v)
    pltpu.async_copy(au.at[iu_v], ru_v, sem).wait()
    pltpu.sync_copy(ru_v, ue.at[pl.ds(u0, UPW)])
    b0 = wid * BPW
    pltpu.sync_copy(bidx.at[pl.ds(b0, BPW)], ib_v)
    pltpu.async_copy(brep.at[ib_v], rb_v, sem).wait()
    pltpu.sync_copy(rb_v, be.at[pl.ds(b0, BPW)])


_lookup_call = functools.partial(
    pl.kernel,
    out_type=(
        jax.ShapeDtypeStruct((BATCH, EMB), f32),
        jax.ShapeDtypeStruct((2 * BATCH, EMB), f32),
    ),
    mesh=_MESH,
    compiler_params=_SC_PARAMS,
    scratch_types=[
        pltpu.VMEM((UPW,), i32),
        pltpu.VMEM((BPW,), i32),
        pltpu.VMEM((UPW, EMB), f32),
        pltpu.VMEM((BPW, EMB), f32),
        pltpu.SemaphoreType.DMA,
    ],
)(_lookup_body)


def _loss_body(u_ref, pos_ref, neg_ref, out_ref):
    u = u_ref[...]
    x = jnp.sum(u * (neg_ref[...] - pos_ref[...]), axis=1)
    sp = jnp.maximum(x, 0.0) + jnp.log(1.0 + jnp.exp(-jnp.abs(x)))
    out_ref[...] = jnp.broadcast_to(jnp.mean(sp), (1, 1))


def _loss_call(ue, pos, neg):
    return pl.pallas_call(
        _loss_body,
        out_shape=jax.ShapeDtypeStruct((1, 1), f32),
    )(ue, pos, neg)


def kernel(users_feature, items_feature, bundles_feature,
           ui_u, ui_i, bi_b, bi_i, users, bundles):
    ui_u = ui_u.astype(i32)
    ui_i = ui_i.astype(i32)
    bi_b = bi_b.astype(i32)
    bi_i = bi_i.astype(i32)

    ones_c = jnp.ones((CD,), f32)

    deg_u, deg_i, bsz = _deg_call(ui_u, ui_i, bi_b, ones_c)
    ru = 1.0 / (jnp.sqrt(deg_u[:NU]) + 1e-8)
    ri = 1.0 / (jnp.sqrt(deg_i[:NI]) + 1e-8)
    binv = 1.0 / (bsz[:NB] + 1e-8)

    ru2 = jnp.pad(ru * ru, (0, NUP - NU))
    ri2 = jnp.pad(ri * ri, (0, NUP - NI))
    h1up, h1ip, t0u, t0i, t1u, t1i = _spmm_scaled_call(
        users_feature[:, :HEMB] * ru[:, None],
        items_feature[:, :HEMB] * ri[:, None],
        users_feature[:, HEMB:] * ru[:, None],
        items_feature[:, HEMB:] * ri[:, None],
        ui_u, ui_i, ru2, ri2)
    h1u, h1i = h1up[:NU], h1ip[:NI]
    h2up, h2ip = _spmm_call(t0u, t0i, t1u, t1i, ui_u, ui_i)
    h2u, h2i = h2up[:NU], h2ip[:NI]

    au = (users_feature + (h1u + h2u) * ru[:, None]) / 3.0
    ai = (items_feature + (h1i + h2i) * ri[:, None]) / 3.0

    hb = _bi_call(ai, bi_b, bi_i)
    brep = (hb[:NB] + hb[NBA:NBA + NB]) * binv[:, None]

    ue, be = _lookup_call(au, brep,
                          users.reshape(-1).astype(i32),
                          bundles.reshape(-1).astype(i32))
    be = be.reshape(BATCH, 2, EMB)
    loss = _loss_call(ue, be[:, 0, :], be[:, 1, :])
    return (loss[0, 0], jnp.zeros(1, f32))
